# baseline (speedup 1.0000x reference)
import functools

import jax
import jax.numpy as jnp
from jax import lax
from jax.experimental import pallas as pl
from jax.experimental.pallas import tpu as pltpu
from jax.experimental.pallas import tpu_sc as plsc

NW = 32
CHUNK = 10000
NBUF = 3


def _fill_1d(ref, total, value):
    v = jnp.full((16,), value, jnp.float32)

    def body(i, _):
        ref[pl.ds(i * 16, 16)] = v
        return 0

    lax.fori_loop(0, total // 16, body, 0)


def _worker_id():
    return lax.axis_index("s") * 2 + lax.axis_index("c")


CHUNKD = 20000


def _deg_pass(dst, n_pad):
    nch = dst.shape[0] // CHUNKD
    ncw = nch // NW
    stripe = n_pad // 16
    mesh = plsc.VectorSubcoreMesh(core_axis_name="c", subcore_axis_name="s")

    @functools.partial(
        pl.kernel,
        out_type=jax.ShapeDtypeStruct((2, n_pad), jnp.float32),
        mesh=mesh,
        scratch_types=[
            [pltpu.VMEM((CHUNKD,), jnp.int32)] * NBUF,
            pltpu.VMEM((CHUNKD,), jnp.float32),
            pltpu.VMEM((stripe,), jnp.float32),
            pltpu.VMEM_SHARED((n_pad,), jnp.float32),
            [pltpu.SemaphoreType.DMA] * NBUF,
            [pltpu.SemaphoreType.DMA] * NBUF,
        ],
    )
    def body(dst_hbm, out_hbm, didx, ones_v, zer_v, acc, lsem, ssem):
        cid = lax.axis_index("c")
        sid = lax.axis_index("s")
        w = _worker_id()
        _fill_1d(ones_v, CHUNKD, 1.0)
        _fill_1d(zer_v, stripe, 0.0)
        pltpu.sync_copy(zer_v, acc.at[pl.ds(sid * stripe, stripe)])
        plsc.subcore_barrier()

        ldp = [None] * NBUF
        scp = [None] * NBUF

        def loads(j):
            s = j % NBUF
            ci = w + j * NW
            ldp[s] = pltpu.async_copy(
                dst_hbm.at[pl.ds(ci * CHUNKD, CHUNKD)], didx[s], lsem[s])

        loads(0)
        if ncw > 1:
            loads(1)
        for j in range(ncw):
            s = j % NBUF
            ldp[s].wait()
            if j >= 1:
                scp[(j - 1) % NBUF].wait()
            if j + 2 < ncw:
                loads(j + 2)
            scp[s] = pltpu.async_copy(ones_v, acc.at[didx[s]], ssem[s],
                                      add=True)
        scp[(ncw - 1) % NBUF].wait()
        plsc.subcore_barrier()
        pltpu.sync_copy(acc.at[pl.ds(sid * stripe, stripe)],
                        out_hbm.at[cid, pl.ds(sid * stripe, stripe)])

    return body(dst)


def _edge_pass1(src, dst, tab, n_pad):
    nch = dst.shape[0] // CHUNK
    ncw = nch // NW
    stripe = n_pad // 16
    mesh = plsc.VectorSubcoreMesh(core_axis_name="c", subcore_axis_name="s")

    @functools.partial(
        pl.kernel,
        out_type=jax.ShapeDtypeStruct((2, n_pad), jnp.float32),
        mesh=mesh,
        scratch_types=[
            [pltpu.VMEM((CHUNK,), jnp.int32)] * NBUF,
            [pltpu.VMEM((CHUNK,), jnp.int32)] * NBUF,
            [pltpu.VMEM((CHUNK,), jnp.float32)] * NBUF,
            pltpu.VMEM((stripe,), jnp.float32),
            pltpu.VMEM_SHARED((n_pad,), jnp.float32),
            pltpu.VMEM_SHARED((n_pad,), jnp.float32),
            [pltpu.SemaphoreType.DMA] * NBUF,
            [pltpu.SemaphoreType.DMA] * NBUF,
            [pltpu.SemaphoreType.DMA] * NBUF,
        ],
    )
    def body(src_hbm, dst_hbm, tab_hbm, out_hbm,
             sidx, didx, val, zer_v, acc, tab_sh, lsem, gsem, ssem):
        cid = lax.axis_index("c")
        sid = lax.axis_index("s")
        w = _worker_id()
        _fill_1d(zer_v, stripe, 0.0)
        pltpu.sync_copy(zer_v, acc.at[pl.ds(sid * stripe, stripe)])
        pltpu.sync_copy(tab_hbm.at[pl.ds(sid * stripe, stripe)],
                        tab_sh.at[pl.ds(sid * stripe, stripe)])
        plsc.subcore_barrier()

        ldp = [None] * NBUF
        scp = [None] * NBUF

        def loads(j):
            s = j % NBUF
            ci = w + j * NW
            a = pltpu.async_copy(
                src_hbm.at[pl.ds(ci * CHUNK, CHUNK)], sidx[s], lsem[s])
            b = pltpu.async_copy(
                dst_hbm.at[pl.ds(ci * CHUNK, CHUNK)], didx[s], lsem[s])
            ldp[s] = (a, b)

        loads(0)
        if ncw > 1:
            loads(1)
        for j in range(ncw):
            s = j % NBUF
            ldp[s][0].wait()
            ldp[s][1].wait()
            gat = pltpu.async_copy(tab_sh.at[sidx[s]], val[s], gsem[s])
            if j >= 1:
                scp[(j - 1) % NBUF].wait()
            if j + 2 < ncw:
                loads(j + 2)
            gat.wait()
            scp[s] = pltpu.async_copy(val[s], acc.at[didx[s]], ssem[s],
                                      add=True)
        scp[(ncw - 1) % NBUF].wait()
        plsc.subcore_barrier()
        pltpu.sync_copy(acc.at[pl.ds(sid * stripe, stripe)],
                        out_hbm.at[cid, pl.ds(sid * stripe, stripe)])

    return body(src, dst, tab)


CHUNK2 = 10000


def _edge_pass2(src, dst, tab_packed, n_pad):
    nch = dst.shape[0] // CHUNK2
    ncw = nch // NW
    stripe = n_pad // 16
    mesh = plsc.VectorSubcoreMesh(core_axis_name="c", subcore_axis_name="s")

    @functools.partial(
        pl.kernel,
        out_type=[jax.ShapeDtypeStruct((2, n_pad), jnp.float32)] * 2,
        mesh=mesh,
        compiler_params=pltpu.CompilerParams(needs_layout_passes=False),
        scratch_types=[
            [pltpu.VMEM((CHUNK2,), jnp.int32)] * 2,
            [pltpu.VMEM((CHUNK2,), jnp.int32)] * 2,
            [pltpu.VMEM((CHUNK2,), jnp.int32)] * 2,
            [pltpu.VMEM((CHUNK2,), jnp.float32)] * 2,
            [pltpu.VMEM((CHUNK2,), jnp.float32)] * 2,
            pltpu.VMEM_SHARED((n_pad,), jnp.float32),
            pltpu.VMEM_SHARED((n_pad,), jnp.float32),
            pltpu.VMEM_SHARED((n_pad,), jnp.int32),
            [pltpu.SemaphoreType.DMA] * 2,
            [pltpu.SemaphoreType.DMA] * 2,
            [pltpu.SemaphoreType.DMA] * 2,
            [pltpu.SemaphoreType.DMA] * 2,
        ],
    )
    def body(src_hbm, dst_hbm, tabp_hbm, out0_hbm, out1_hbm,
             sidx, didx, valp, val0, val1, acc0, acc1, tabp_sh,
             lsem, gsem, s0sem, s1sem):
        cid = lax.axis_index("c")
        sid = lax.axis_index("s")
        w = _worker_id()
        sl = pl.ds(sid * stripe, stripe)
        _fill_1d(val0[0], stripe, 0.0)
        zsl = val0[0].at[pl.ds(0, stripe)]
        pltpu.sync_copy(zsl, acc0.at[sl])
        pltpu.sync_copy(zsl, acc1.at[sl])
        pltpu.sync_copy(tabp_hbm.at[sl], tabp_sh.at[sl])
        plsc.subcore_barrier()

        ldp = [None] * 2
        sc0 = [None] * 2
        sc1 = [None] * 2
        himask = jnp.full((16,), -65536, jnp.int32)
        sh16 = jnp.full((16,), 16, jnp.int32)

        def loads(j):
            s = j % 2
            ci = w + j * NW
            a = pltpu.async_copy(
                src_hbm.at[pl.ds(ci * CHUNK2, CHUNK2)], sidx[s], lsem[s])
            b = pltpu.async_copy(
                dst_hbm.at[pl.ds(ci * CHUNK2, CHUNK2)], didx[s], lsem[s])
            ldp[s] = (a, b)

        def unpack(s):
            def ub(i, _):
                wv = valp[s][pl.ds(i * 16, 16)]
                val0[s][pl.ds(i * 16, 16)] = plsc.bitcast(
                    lax.shift_left(wv, sh16), jnp.float32)
                val1[s][pl.ds(i * 16, 16)] = plsc.bitcast(
                    wv & himask, jnp.float32)
                return 0
            lax.fori_loop(0, CHUNK2 // 16, ub, 0, unroll=4)

        loads(0)
        for j in range(ncw):
            s = j % 2
            o = 1 - s
            ldp[s][0].wait()
            ldp[s][1].wait()
            if sc0[s] is not None:
                sc0[s].wait()
                sc1[s].wait()
            g = pltpu.async_copy(tabp_sh.at[sidx[s]], valp[s], gsem[s])
            if sc0[o] is not None:
                sc0[o].wait()
                sc1[o].wait()
                sc0[o] = sc1[o] = None
            if j + 1 < ncw:
                loads(j + 1)
            g.wait()
            unpack(s)
            sc0[s] = pltpu.async_copy(val0[s], acc0.at[didx[s]], s0sem[s],
                                      add=True)
            sc1[s] = pltpu.async_copy(val1[s], acc1.at[didx[s]], s1sem[s],
                                      add=True)
        sc0[(ncw - 1) % 2].wait()
        sc1[(ncw - 1) % 2].wait()
        plsc.subcore_barrier()
        pltpu.sync_copy(acc0.at[sl], out0_hbm.at[cid, sl])
        pltpu.sync_copy(acc1.at[sl], out1_hbm.at[cid, sl])

    return body(src, dst, tab_packed)


def _tc_node1(degp2, x2):
    def body(degp_ref, x_ref, dinv_ref, y_ref):
        deg = degp_ref[0] + degp_ref[1] + 1.0
        dinv = lax.rsqrt(deg)
        dinv_ref[...] = dinv
        y_ref[...] = dinv * x_ref[...]

    return pl.pallas_call(
        body,
        out_shape=[jax.ShapeDtypeStruct(x2.shape, jnp.float32)] * 2,
    )(degp2, x2)


def _tc_node2(tp2, dinv2, y2, W1, b1, W2):
    def body(tp_ref, dinv_ref, y_ref, w1_ref, b1_ref, w2_ref,
             z0_ref, z1_ref, zp_ref):
        dinv = dinv_ref[...]
        s1 = dinv * (tp_ref[0] + tp_ref[1] + y_ref[...])
        h0 = jnp.zeros_like(s1)
        h1 = jnp.zeros_like(s1)
        for k in range(16):
            a = jnp.maximum(s1 * w1_ref[0, k] + b1_ref[k], 0.0)
            h0 += a * w2_ref[k, 0]
            h1 += a * w2_ref[k, 1]
        z0 = dinv * h0
        z1 = dinv * h1
        z0_ref[...] = z0
        z1_ref[...] = z1
        u0 = lax.bitcast_convert_type(z0, jnp.uint32)
        u1 = lax.bitcast_convert_type(z1, jnp.uint32)
        r0 = (u0 + jnp.uint32(0x7FFF) + ((u0 >> 16) & 1)) >> 16
        r1 = (u1 + jnp.uint32(0x7FFF) + ((u1 >> 16) & 1)) >> 16
        zp_ref[...] = lax.bitcast_convert_type(r0 | (r1 << 16), jnp.int32)

    return pl.pallas_call(
        body,
        out_shape=[jax.ShapeDtypeStruct(dinv2.shape, jnp.float32)] * 2
        + [jax.ShapeDtypeStruct(dinv2.shape, jnp.int32)],
        in_specs=[
            pl.BlockSpec(memory_space=pltpu.MemorySpace.VMEM),
            pl.BlockSpec(memory_space=pltpu.MemorySpace.VMEM),
            pl.BlockSpec(memory_space=pltpu.MemorySpace.VMEM),
            pl.BlockSpec(memory_space=pltpu.SMEM),
            pl.BlockSpec(memory_space=pltpu.SMEM),
            pl.BlockSpec(memory_space=pltpu.SMEM),
        ],
    )(tp2, dinv2, y2, W1, b1, W2)


def _tc_node3(t02, t12, z0, z1, dinv2, b2):
    def body(t0_ref, t1_ref, z0_ref, z1_ref, dinv_ref, b2_ref, o0_ref, o1_ref):
        dinv = dinv_ref[...]
        s0 = dinv * (t0_ref[0] + t0_ref[1] + z0_ref[...]) + b2_ref[0]
        s1 = dinv * (t1_ref[0] + t1_ref[1] + z1_ref[...]) + b2_ref[1]
        m = jnp.maximum(s0, s1)
        lse = m + jnp.log(jnp.exp(s0 - m) + jnp.exp(s1 - m))
        o0_ref[...] = s0 - lse
        o1_ref[...] = s1 - lse

    return pl.pallas_call(
        body,
        out_shape=[jax.ShapeDtypeStruct(dinv2.shape, jnp.float32)] * 2,
        in_specs=[
            pl.BlockSpec(memory_space=pltpu.MemorySpace.VMEM),
            pl.BlockSpec(memory_space=pltpu.MemorySpace.VMEM),
            pl.BlockSpec(memory_space=pltpu.MemorySpace.VMEM),
            pl.BlockSpec(memory_space=pltpu.MemorySpace.VMEM),
            pl.BlockSpec(memory_space=pltpu.MemorySpace.VMEM),
            pl.BlockSpec(memory_space=pltpu.SMEM),
        ],
    )(t02, t12, z0, z1, dinv2, b2)


def kernel(x, edge_index, W1, b1, W2, b2):
    n = x.shape[0]
    r = -(-n // 128)
    r = -(-r // 8) * 8
    n_pad = r * 128

    src = edge_index[0]
    dst = edge_index[1]
    x2 = jnp.pad(x[:, 0], (0, n_pad - n)).reshape(r, 128)

    degp = _deg_pass(dst, n_pad)
    dinv2, y2 = _tc_node1(degp.reshape(2, r, 128), x2)

    tp = _edge_pass1(src, dst, y2.reshape(-1), n_pad)
    z0, z1, zp = _tc_node2(tp.reshape(2, r, 128), dinv2, y2, W1, b1, W2)

    t0, t1 = _edge_pass2(src, dst, zp.reshape(-1), n_pad)
    o0, o1 = _tc_node3(t0.reshape(2, r, 128), t1.reshape(2, r, 128),
                       z0, z1, dinv2, b2)

    return jnp.stack([o0.reshape(-1)[:n], o1.reshape(-1)[:n]], axis=1)


# speedup vs baseline: 1.1375x; 1.1375x over previous
import functools

import jax
import jax.numpy as jnp
from jax import lax
from jax.experimental import pallas as pl
from jax.experimental.pallas import tpu as pltpu
from jax.experimental.pallas import tpu_sc as plsc

NW = 32
CHUNK = 10000
NBUF = 3


def _fill_1d(ref, total, value):
    v = jnp.full((16,), value, jnp.float32)

    def body(i, _):
        ref[pl.ds(i * 16, 16)] = v
        return 0

    lax.fori_loop(0, total // 16, body, 0)


def _worker_id():
    return lax.axis_index("s") * 2 + lax.axis_index("c")


CHUNKD = 20000


def _deg_pass(dst, n_pad):
    nch = dst.shape[0] // CHUNKD
    ncw = nch // NW
    stripe = n_pad // 16
    mesh = plsc.VectorSubcoreMesh(core_axis_name="c", subcore_axis_name="s")

    @functools.partial(
        pl.kernel,
        out_type=jax.ShapeDtypeStruct((2, n_pad), jnp.float32),
        mesh=mesh,
        scratch_types=[
            [pltpu.VMEM((CHUNKD,), jnp.int32)] * NBUF,
            pltpu.VMEM((CHUNKD,), jnp.float32),
            pltpu.VMEM((stripe,), jnp.float32),
            pltpu.VMEM_SHARED((n_pad,), jnp.float32),
            [pltpu.SemaphoreType.DMA] * NBUF,
            [pltpu.SemaphoreType.DMA] * NBUF,
        ],
    )
    def body(dst_hbm, out_hbm, didx, ones_v, zer_v, acc, lsem, ssem):
        cid = lax.axis_index("c")
        sid = lax.axis_index("s")
        w = _worker_id()
        _fill_1d(ones_v, CHUNKD, 1.0)
        _fill_1d(zer_v, stripe, 0.0)
        pltpu.sync_copy(zer_v, acc.at[pl.ds(sid * stripe, stripe)])
        plsc.subcore_barrier()

        ldp = [None] * NBUF
        scp = [None] * NBUF

        def loads(j):
            s = j % NBUF
            ci = w + j * NW
            ldp[s] = pltpu.async_copy(
                dst_hbm.at[pl.ds(ci * CHUNKD, CHUNKD)], didx[s], lsem[s])

        loads(0)
        if ncw > 1:
            loads(1)
        for j in range(ncw):
            s = j % NBUF
            ldp[s].wait()
            if j >= 1:
                scp[(j - 1) % NBUF].wait()
            if j + 2 < ncw:
                loads(j + 2)
            scp[s] = pltpu.async_copy(ones_v, acc.at[didx[s]], ssem[s],
                                      add=True)
        scp[(ncw - 1) % NBUF].wait()
        plsc.subcore_barrier()
        pltpu.sync_copy(acc.at[pl.ds(sid * stripe, stripe)],
                        out_hbm.at[cid, pl.ds(sid * stripe, stripe)])

    return body(dst)


def _edge_pass1(src, dst, tab, n_pad):
    nch = dst.shape[0] // CHUNK
    ncw = nch // NW
    stripe = n_pad // 16
    mesh = plsc.VectorSubcoreMesh(core_axis_name="c", subcore_axis_name="s")

    @functools.partial(
        pl.kernel,
        out_type=jax.ShapeDtypeStruct((2, n_pad), jnp.float32),
        mesh=mesh,
        scratch_types=[
            [pltpu.VMEM((CHUNK,), jnp.int32)] * NBUF,
            [pltpu.VMEM((CHUNK,), jnp.int32)] * NBUF,
            [pltpu.VMEM((CHUNK,), jnp.float32)] * NBUF,
            pltpu.VMEM((stripe,), jnp.float32),
            pltpu.VMEM_SHARED((n_pad,), jnp.float32),
            pltpu.VMEM_SHARED((n_pad,), jnp.float32),
            [pltpu.SemaphoreType.DMA] * NBUF,
            [pltpu.SemaphoreType.DMA] * NBUF,
            [pltpu.SemaphoreType.DMA] * NBUF,
        ],
    )
    def body(src_hbm, dst_hbm, tab_hbm, out_hbm,
             sidx, didx, val, zer_v, acc, tab_sh, lsem, gsem, ssem):
        cid = lax.axis_index("c")
        sid = lax.axis_index("s")
        w = _worker_id()
        _fill_1d(zer_v, stripe, 0.0)
        pltpu.sync_copy(zer_v, acc.at[pl.ds(sid * stripe, stripe)])
        pltpu.sync_copy(tab_hbm.at[pl.ds(sid * stripe, stripe)],
                        tab_sh.at[pl.ds(sid * stripe, stripe)])
        plsc.subcore_barrier()

        ldp = [None] * NBUF
        scp = [None] * NBUF

        def loads(j):
            s = j % NBUF
            ci = w + j * NW
            a = pltpu.async_copy(
                src_hbm.at[pl.ds(ci * CHUNK, CHUNK)], sidx[s], lsem[s])
            b = pltpu.async_copy(
                dst_hbm.at[pl.ds(ci * CHUNK, CHUNK)], didx[s], lsem[s])
            ldp[s] = (a, b)

        loads(0)
        if ncw > 1:
            loads(1)
        for j in range(ncw):
            s = j % NBUF
            ldp[s][0].wait()
            ldp[s][1].wait()
            gat = pltpu.async_copy(tab_sh.at[sidx[s]], val[s], gsem[s])
            if j >= 1:
                scp[(j - 1) % NBUF].wait()
            if j + 2 < ncw:
                loads(j + 2)
            gat.wait()
            scp[s] = pltpu.async_copy(val[s], acc.at[didx[s]], ssem[s],
                                      add=True)
        scp[(ncw - 1) % NBUF].wait()
        plsc.subcore_barrier()
        pltpu.sync_copy(acc.at[pl.ds(sid * stripe, stripe)],
                        out_hbm.at[cid, pl.ds(sid * stripe, stripe)])

    return body(src, dst, tab)


CHUNK2 = 10000


def _edge_pass2(src, dst, tab0, tab1, n_pad):
    nch = dst.shape[0] // CHUNK2
    ncw = nch // NW
    stripe = n_pad // 16
    mesh = plsc.VectorSubcoreMesh(core_axis_name="c", subcore_axis_name="s")

    @functools.partial(
        pl.kernel,
        out_type=[jax.ShapeDtypeStruct((2, n_pad), jnp.float32)] * 2,
        mesh=mesh,
        scratch_types=[
            [pltpu.VMEM((CHUNK2,), jnp.int32)] * 2,
            [pltpu.VMEM((CHUNK2,), jnp.int32)] * 2,
            [pltpu.VMEM((CHUNK2,), jnp.float32)] * 2,
            [pltpu.VMEM((CHUNK2,), jnp.float32)] * 2,
            pltpu.VMEM_SHARED((n_pad,), jnp.float32),
            pltpu.VMEM_SHARED((n_pad,), jnp.float32),
            pltpu.VMEM_SHARED((n_pad,), jnp.float32),
            pltpu.VMEM_SHARED((n_pad,), jnp.float32),
            [pltpu.SemaphoreType.DMA] * 2,
            [pltpu.SemaphoreType.DMA] * 2,
            [pltpu.SemaphoreType.DMA] * 2,
            [pltpu.SemaphoreType.DMA] * 2,
            [pltpu.SemaphoreType.DMA] * 2,
        ],
    )
    def body(src_hbm, dst_hbm, tab0_hbm, tab1_hbm, out0_hbm, out1_hbm,
             sidx, didx, val0, val1, acc0, acc1, tab0_sh, tab1_sh,
             lsem, g0sem, g1sem, s0sem, s1sem):
        cid = lax.axis_index("c")
        sid = lax.axis_index("s")
        w = _worker_id()
        sl = pl.ds(sid * stripe, stripe)
        _fill_1d(val0[0], stripe, 0.0)
        zsl = val0[0].at[pl.ds(0, stripe)]
        pltpu.sync_copy(zsl, acc0.at[sl])
        pltpu.sync_copy(zsl, acc1.at[sl])
        pltpu.sync_copy(tab0_hbm.at[sl], tab0_sh.at[sl])
        pltpu.sync_copy(tab1_hbm.at[sl], tab1_sh.at[sl])
        plsc.subcore_barrier()

        ldp = [None] * 2
        sc0 = [None] * 2
        sc1 = [None] * 2

        def loads(j):
            s = j % 2
            ci = w + j * NW
            a = pltpu.async_copy(
                src_hbm.at[pl.ds(ci * CHUNK2, CHUNK2)], sidx[s], lsem[s])
            b = pltpu.async_copy(
                dst_hbm.at[pl.ds(ci * CHUNK2, CHUNK2)], didx[s], lsem[s])
            ldp[s] = (a, b)

        loads(0)
        for j in range(ncw):
            s = j % 2
            o = 1 - s
            ldp[s][0].wait()
            ldp[s][1].wait()
            if sc0[s] is not None:
                sc0[s].wait()
                sc1[s].wait()
            g0 = pltpu.async_copy(tab0_sh.at[sidx[s]], val0[s], g0sem[s])
            g1 = pltpu.async_copy(tab1_sh.at[sidx[s]], val1[s], g1sem[s])
            if sc0[o] is not None:
                sc0[o].wait()
                sc1[o].wait()
                sc0[o] = sc1[o] = None
            if j + 1 < ncw:
                loads(j + 1)
            g0.wait()
            sc0[s] = pltpu.async_copy(val0[s], acc0.at[didx[s]], s0sem[s],
                                      add=True)
            g1.wait()
            sc1[s] = pltpu.async_copy(val1[s], acc1.at[didx[s]], s1sem[s],
                                      add=True)
        sc0[(ncw - 1) % 2].wait()
        sc1[(ncw - 1) % 2].wait()
        plsc.subcore_barrier()
        pltpu.sync_copy(acc0.at[sl], out0_hbm.at[cid, sl])
        pltpu.sync_copy(acc1.at[sl], out1_hbm.at[cid, sl])

    return body(src, dst, tab0, tab1)


def _tc_node1(degp2, x2):
    def body(degp_ref, x_ref, dinv_ref, y_ref):
        deg = degp_ref[0] + degp_ref[1] + 1.0
        dinv = lax.rsqrt(deg)
        dinv_ref[...] = dinv
        y_ref[...] = dinv * x_ref[...]

    return pl.pallas_call(
        body,
        out_shape=[jax.ShapeDtypeStruct(x2.shape, jnp.float32)] * 2,
    )(degp2, x2)


def _tc_node2(tp2, dinv2, y2, W1, b1, W2):
    def body(tp_ref, dinv_ref, y_ref, w1_ref, b1_ref, w2_ref, z0_ref, z1_ref):
        dinv = dinv_ref[...]
        s1 = dinv * (tp_ref[0] + tp_ref[1] + y_ref[...])
        h0 = jnp.zeros_like(s1)
        h1 = jnp.zeros_like(s1)
        for k in range(16):
            a = jnp.maximum(s1 * w1_ref[0, k] + b1_ref[k], 0.0)
            h0 += a * w2_ref[k, 0]
            h1 += a * w2_ref[k, 1]
        z0_ref[...] = dinv * h0
        z1_ref[...] = dinv * h1

    return pl.pallas_call(
        body,
        out_shape=[jax.ShapeDtypeStruct(dinv2.shape, jnp.float32)] * 2,
        in_specs=[
            pl.BlockSpec(memory_space=pltpu.MemorySpace.VMEM),
            pl.BlockSpec(memory_space=pltpu.MemorySpace.VMEM),
            pl.BlockSpec(memory_space=pltpu.MemorySpace.VMEM),
            pl.BlockSpec(memory_space=pltpu.SMEM),
            pl.BlockSpec(memory_space=pltpu.SMEM),
            pl.BlockSpec(memory_space=pltpu.SMEM),
        ],
    )(tp2, dinv2, y2, W1, b1, W2)


def _tc_node3(t02, t12, z0, z1, dinv2, b2):
    def body(t0_ref, t1_ref, z0_ref, z1_ref, dinv_ref, b2_ref, o0_ref, o1_ref):
        dinv = dinv_ref[...]
        s0 = dinv * (t0_ref[0] + t0_ref[1] + z0_ref[...]) + b2_ref[0]
        s1 = dinv * (t1_ref[0] + t1_ref[1] + z1_ref[...]) + b2_ref[1]
        m = jnp.maximum(s0, s1)
        lse = m + jnp.log(jnp.exp(s0 - m) + jnp.exp(s1 - m))
        o0_ref[...] = s0 - lse
        o1_ref[...] = s1 - lse

    return pl.pallas_call(
        body,
        out_shape=[jax.ShapeDtypeStruct(dinv2.shape, jnp.float32)] * 2,
        in_specs=[
            pl.BlockSpec(memory_space=pltpu.MemorySpace.VMEM),
            pl.BlockSpec(memory_space=pltpu.MemorySpace.VMEM),
            pl.BlockSpec(memory_space=pltpu.MemorySpace.VMEM),
            pl.BlockSpec(memory_space=pltpu.MemorySpace.VMEM),
            pl.BlockSpec(memory_space=pltpu.MemorySpace.VMEM),
            pl.BlockSpec(memory_space=pltpu.SMEM),
        ],
    )(t02, t12, z0, z1, dinv2, b2)


def kernel(x, edge_index, W1, b1, W2, b2):
    n = x.shape[0]
    r = -(-n // 128)
    r = -(-r // 8) * 8
    n_pad = r * 128

    src = edge_index[0]
    dst = edge_index[1]
    x2 = jnp.pad(x[:, 0], (0, n_pad - n)).reshape(r, 128)

    degp = _deg_pass(dst, n_pad)
    dinv2, y2 = _tc_node1(degp.reshape(2, r, 128), x2)

    tp = _edge_pass1(src, dst, y2.reshape(-1), n_pad)
    z0, z1 = _tc_node2(tp.reshape(2, r, 128), dinv2, y2, W1, b1, W2)

    t0, t1 = _edge_pass2(src, dst, z0.reshape(-1), z1.reshape(-1), n_pad)
    o0, o1 = _tc_node3(t0.reshape(2, r, 128), t1.reshape(2, r, 128),
                       z0, z1, dinv2, b2)

    return jnp.stack([o0.reshape(-1)[:n], o1.reshape(-1)[:n]], axis=1)
